# baseline (speedup 1.0000x reference)
import functools

import jax
import jax.numpy as jnp
from jax import lax
from jax.experimental import pallas as pl
from jax.experimental.pallas import tpu as pltpu
from jax.experimental.pallas import tpu_sc as plsc

B = 16384
E = 128
N = 2048
R = 8

NC = 2
NS = 16
LN = 16
NW = NC * NS
BPW = B // NW
GROUPS = BPW // LN
IDX_COLS = 128
NCHUNK = (BPW * R) // IDX_COLS


def _sc_gather_sum(a_hbm, id1_hbm, id2_hbm, s_hbm,
                   id1_v, id2_v, idx_v, vals_v, out_v, idsem, sem):
    wid = lax.axis_index("s") * NC + lax.axis_index("c")
    base = wid * BPW
    c1 = pltpu.async_copy(id1_hbm.at[pl.ds(base, BPW)], id1_v, idsem)
    c2 = pltpu.async_copy(id2_hbm.at[pl.ds(base, BPW)], id2_v, idsem)
    c1.wait()
    c2.wait()

    lane_iota = lax.iota(jnp.int32, LN)
    copies = []
    for g in range(GROUPS):
        i = id1_v[pl.ds(g * LN, LN)]
        j = id2_v[pl.ds(g * LN, LN)]
        w = (((i >> 3) << 14) + ((j >> 7) << 10)
             + ((i & 7) << 7) + (j & 127))
        for rel in range(R):
            plsc.store_scatter(idx_v.at[g], [lane_iota * R + rel],
                               w + rel * (N * N))
        dst = vals_v.at[pl.ds(g * IDX_COLS, IDX_COLS)]
        copies.append(pltpu.async_copy(a_hbm.at[idx_v.at[g]], dst, sem))
    for cp in copies:
        cp.wait()

    for g in range(GROUPS):
        pbase = (lane_iota + g * LN) * R
        acc = plsc.load_gather(vals_v, [pbase])
        for rel in range(1, R):
            acc = acc + plsc.load_gather(vals_v, [pbase + rel])
        out_v[pl.ds(g * LN, LN)] = acc
    pltpu.sync_copy(out_v, s_hbm.at[pl.ds(base, BPW)])


def _tc_combine(emb1_ref, emb2_ref, w_ref, s_ref, out_ref):
    w1 = w_ref[:, :E]
    w2 = w_ref[:, E:]
    r = (jax.lax.dot_general(w1, emb1_ref[...], (((1,), (1,)), ((), ())),
                             preferred_element_type=jnp.float32)
         + jax.lax.dot_general(w2, emb2_ref[...], (((1,), (1,)), ((), ())),
                               preferred_element_type=jnp.float32))
    out_ref[...] = jnp.maximum(r[0, :], 0.0) * s_ref[...]


def kernel(id1, id2, emb1, emb2, A_list, w):
    id1 = id1.astype(jnp.int32)
    id2 = id2.astype(jnp.int32)

    a_phys = (A_list.reshape(R, N // 8, 8, N // 128, 128)
              .swapaxes(2, 3)
              .reshape(R * N * N))

    mesh = plsc.VectorSubcoreMesh(core_axis_name="c", subcore_axis_name="s",
                                  num_cores=NC, num_subcores=NS)
    s = pl.kernel(
        _sc_gather_sum,
        out_type=jax.ShapeDtypeStruct((B,), jnp.float32),
        mesh=mesh,
        compiler_params=pltpu.CompilerParams(needs_layout_passes=False),
        scratch_types=[
            pltpu.VMEM((BPW,), jnp.int32),
            pltpu.VMEM((BPW,), jnp.int32),
            pltpu.VMEM((NCHUNK, IDX_COLS), jnp.int32),
            pltpu.VMEM((R * BPW,), jnp.float32),
            pltpu.VMEM((BPW,), jnp.float32),
            pltpu.SemaphoreType.DMA,
            pltpu.SemaphoreType.DMA,
        ],
    )(a_phys, id1, id2)

    blk = 2048
    out = pl.pallas_call(
        _tc_combine,
        grid=(B // blk,),
        in_specs=[
            pl.BlockSpec((blk, E), lambda i: (i, 0)),
            pl.BlockSpec((blk, E), lambda i: (i, 0)),
            pl.BlockSpec((1, 2 * E), lambda i: (0, 0)),
            pl.BlockSpec((blk,), lambda i: (i,)),
        ],
        out_specs=pl.BlockSpec((blk,), lambda i: (i,)),
        out_shape=jax.ShapeDtypeStruct((B,), jnp.float32),
    )(emb1, emb2, w, s)
    return out


# speedup vs baseline: 1.2394x; 1.2394x over previous
import functools

import jax
import jax.numpy as jnp
from jax import lax
from jax.experimental import pallas as pl
from jax.experimental.pallas import tpu as pltpu
from jax.experimental.pallas import tpu_sc as plsc

B = 16384
E = 128
N = 2048
R = 8

NC = 2
NS = 16
LN = 16
NW = NC * NS
BPW = B // NW
GROUPS = BPW // LN
IDX_COLS = 128
NCHUNK = (BPW * R) // IDX_COLS


def _sc_gather_sum(a_hbm, id1_hbm, id2_hbm, s_hbm,
                   id1_v, id2_v, idx_v, vals_v, out_v, sem):
    wid = lax.axis_index("s") * NC + lax.axis_index("c")
    base = wid * BPW
    pltpu.sync_copy(id1_hbm.at[pl.ds(base, BPW)], id1_v)
    pltpu.sync_copy(id2_hbm.at[pl.ds(base, BPW)], id2_v)

    for g in range(GROUPS):
        i = id1_v[pl.ds(g * LN, LN)]
        j = id2_v[pl.ds(g * LN, LN)]
        w = (((i >> 3) << 14) + ((j >> 7) << 10)
             + ((i & 7) << 7) + (j & 127))
        for rel in range(R):
            pos = rel * BPW + g * LN
            idx_v[pos // IDX_COLS, pl.ds(pos % IDX_COLS, LN)] = w + rel * (N * N)

    copies = []
    for c in range(NCHUNK):
        pos = c * IDX_COLS
        dst = vals_v.at[pos // BPW, pl.ds(pos % BPW, IDX_COLS)]
        copies.append(pltpu.async_copy(a_hbm.at[idx_v.at[c]], dst, sem))
    for cp in copies:
        cp.wait()

    for g in range(GROUPS):
        acc = vals_v[0, pl.ds(g * LN, LN)]
        for rel in range(1, R):
            acc = acc + vals_v[rel, pl.ds(g * LN, LN)]
        out_v[pl.ds(g * LN, LN)] = acc
    pltpu.sync_copy(out_v, s_hbm.at[pl.ds(base, BPW)])


def _tc_matvec(emb1_ref, emb2_ref, w_ref, r_ref):
    w1 = w_ref[:, :E]
    w2 = w_ref[:, E:]
    r = (jax.lax.dot_general(w1, emb1_ref[...], (((1,), (1,)), ((), ())),
                             preferred_element_type=jnp.float32)
         + jax.lax.dot_general(w2, emb2_ref[...], (((1,), (1,)), ((), ())),
                               preferred_element_type=jnp.float32))
    r_ref[...] = jnp.maximum(r[0, :], 0.0)


def _tc_mul(r_ref, s_ref, out_ref):
    out_ref[...] = r_ref[...] * s_ref[...]


def kernel(id1, id2, emb1, emb2, A_list, w):
    id1 = id1.astype(jnp.int32)
    id2 = id2.astype(jnp.int32)

    a_phys = (A_list.reshape(R, N // 8, 8, N // 128, 128)
              .swapaxes(2, 3)
              .reshape(R * N * N))

    mesh = plsc.VectorSubcoreMesh(core_axis_name="c", subcore_axis_name="s",
                                  num_cores=NC, num_subcores=NS)
    s = pl.kernel(
        _sc_gather_sum,
        out_type=jax.ShapeDtypeStruct((B,), jnp.float32),
        mesh=mesh,
        scratch_types=[
            pltpu.VMEM((BPW,), jnp.int32),
            pltpu.VMEM((BPW,), jnp.int32),
            pltpu.VMEM((NCHUNK, IDX_COLS), jnp.int32),
            pltpu.VMEM((R, BPW), jnp.float32),
            pltpu.VMEM((BPW,), jnp.float32),
            pltpu.SemaphoreType.DMA,
        ],
    )(a_phys, id1, id2)

    blk = 2048
    r = pl.pallas_call(
        _tc_matvec,
        grid=(B // blk,),
        in_specs=[
            pl.BlockSpec((blk, E), lambda i: (i, 0)),
            pl.BlockSpec((blk, E), lambda i: (i, 0)),
            pl.BlockSpec((1, 2 * E), lambda i: (0, 0)),
        ],
        out_specs=pl.BlockSpec((blk,), lambda i: (i,)),
        out_shape=jax.ShapeDtypeStruct((B,), jnp.float32),
    )(emb1, emb2, w)
    out = pl.pallas_call(
        _tc_mul,
        out_shape=jax.ShapeDtypeStruct((B,), jnp.float32),
    )(r, s)
    return out
